# Initial kernel scaffold; baseline (speedup 1.0000x reference)
import jax
import jax.numpy as jnp
from jax.experimental import pallas as pl


def kernel(row_idx, col_idx, edge_vals, conditions_values, Wp1, bp1, Wp2, bp2, Wc1, bc1, Wc2, bc2, Wv1, bv1, Wv2, bv2, Wo1, bo1, Wo2, bo2):
    raise NotImplementedError("write your pallas kernel here")


# speedup vs baseline: 6.5843x; 6.5843x over previous
import functools

import jax
import jax.numpy as jnp
from jax import lax
from jax.experimental import pallas as pl
from jax.experimental.pallas import tpu as pltpu

_F = 64
_VAR = 4096
_CON = 4096
_STEPS = 4
_BLK = 512
_NBLK = _VAR // _BLK


def _pair_norm(x):
    x = x - jnp.mean(x, axis=0, keepdims=True)
    rownorm_mean = jnp.sqrt(1e-06 + jnp.mean(jnp.sum(x * x, axis=1)))
    return x / rownorm_mean


def _leaky(x):
    return jnp.where(x >= 0, x, 0.01 * x)


def _mp_body(A_hbm, cond, noise,
             Wp1, bp1, Wp2, bp2, Wc1, bc1, Wc2, bc2,
             Wv1, bv1, Wv2, bv2, Wo1, bo1, Wo2, bo2,
             o0, o1, o2, o3, ablk, sem):
    outs = (o0, o1, o2, o3)

    def load_blk(b):
        cp = pltpu.make_async_copy(A_hbm.at[pl.ds(b * _BLK, _BLK), :], ablk, sem)
        cp.start()
        cp.wait()
        return ablk[...]

    h = _leaky(cond[...] * Wp1[...][0:1, :] + bp1[...][0:1, :])
    emb = _pair_norm(jnp.dot(h, Wp2[...], preferred_element_type=jnp.float32)
                     + bp2[...][0:1, :])

    constraints = emb
    variables = jnp.ones((_VAR, _F), dtype=jnp.float32)

    Wc1r = Wc1[...]
    cbias = (jnp.dot(emb, Wc1r[_F:2 * _F, :], preferred_element_type=jnp.float32)
             + bc1[...][0:1, :])

    for i in range(_STEPS):
        v2c = jnp.zeros((_CON, _F), dtype=jnp.float32)
        for b in range(_NBLK):
            a = load_blk(b)
            v2c = v2c + lax.dot_general(
                a, variables[b * _BLK:(b + 1) * _BLK, :],
                dimension_numbers=(((0,), (0,)), ((), ())),
                preferred_element_type=jnp.float32)
        hc = _leaky(jnp.dot(constraints, Wc1r[0:_F, :], preferred_element_type=jnp.float32)
                    + jnp.dot(v2c, Wc1r[2 * _F:3 * _F, :], preferred_element_type=jnp.float32)
                    + cbias)
        constraints = _pair_norm(jnp.dot(hc, Wc2[...], preferred_element_type=jnp.float32)
                                 + bc2[...][0:1, :])

        c2v_rows = []
        for b in range(_NBLK):
            a = load_blk(b)
            c2v_rows.append(jnp.dot(a, constraints, preferred_element_type=jnp.float32))
        c2v = jnp.concatenate(c2v_rows, axis=0)
        hv = _leaky(jnp.dot(variables, Wv1[...][0:_F, :], preferred_element_type=jnp.float32)
                    + jnp.dot(c2v, Wv1[...][_F:2 * _F, :], preferred_element_type=jnp.float32)
                    + bv1[...][0:1, :])
        variables = _pair_norm(jnp.dot(hv, Wv2[...], preferred_element_type=jnp.float32)
                               + bv2[...][0:1, :])

        ho = _leaky(jnp.dot(variables, Wo1[...], preferred_element_type=jnp.float32)
                    + bo1[...][0:1, :])
        out = jnp.sum(ho * Wo2[...][:, 0][None, :], axis=1, keepdims=True) + bo2[...][0, 0]
        logits = out + noise[...][i]
        outs[i][...] = 1.0 / (1.0 + jnp.exp(-logits))


def _message_passing(A, cond2d, noise, weights):
    out_shape = [jax.ShapeDtypeStruct((_VAR, 1), jnp.float32)] * _STEPS
    fn = pl.pallas_call(
        _mp_body,
        in_specs=[pl.BlockSpec(memory_space=pl.ANY)]
                 + [pl.BlockSpec(memory_space=pltpu.VMEM)] * (2 + len(weights)),
        out_specs=[pl.BlockSpec(memory_space=pltpu.VMEM)] * _STEPS,
        out_shape=out_shape,
        scratch_shapes=[pltpu.VMEM((_BLK, _CON), jnp.float32),
                        pltpu.SemaphoreType.DMA],
    )
    return fn(A, cond2d, noise, *weights)


def kernel(row_idx, col_idx, edge_vals, conditions_values,
           Wp1, bp1, Wp2, bp2, Wc1, bc1, Wc2, bc2,
           Wv1, bv1, Wv2, bv2, Wo1, bo1, Wo2, bo2):
    flat = row_idx.astype(jnp.int32) * _CON + col_idx.astype(jnp.int32)
    A = jnp.zeros((_VAR * _CON,), jnp.float32).at[flat].add(edge_vals)
    A = A.reshape(_VAR, _CON)

    nkey = jax.random.key(42)
    noise = jnp.stack([
        3.0 * jax.random.normal(jax.random.fold_in(nkey, i), (_VAR, 1), dtype=jnp.float32)
        for i in range(_STEPS)])

    weights = (Wp1, bp1.reshape(1, _F), Wp2, bp2.reshape(1, _F),
               Wc1, bc1.reshape(1, _F), Wc2, bc2.reshape(1, _F),
               Wv1, bv1.reshape(1, _F), Wv2, bv2.reshape(1, _F),
               Wo1, bo1.reshape(1, _F), Wo2, bo2.reshape(1, 1))
    outs = _message_passing(A, conditions_values.reshape(_CON, 1), noise, weights)
    return tuple(outs)
